# baseline (speedup 1.0000x reference)
import jax
import jax.numpy as jnp
import numpy as np
from jax.experimental import pallas as pl
from jax.experimental.pallas import tpu as pltpu

UP = 4
CAPS = (34, 38, 29)
D = 56
S = 12


def _classifier_body(x48_ref, w1_ref, b1_ref, w2_ref, b2_ref, w3_ref, b3_ref,
                     w4_ref, b4_ref, w5_ref, b5_ref, fcw_ref, fcb_ref,
                     logits_ref):
    def lrelu(h):
        return jnp.where(h >= 0, h, 0.1 * h)

    h = jnp.dot(x48_ref[...], w1_ref[...],
                preferred_element_type=jnp.float32) + b1_ref[...]
    h = lrelu(h)
    h = lrelu(jnp.dot(h, w2_ref[...],
                      preferred_element_type=jnp.float32) + b2_ref[...])
    h = lrelu(jnp.dot(h, w3_ref[...],
                      preferred_element_type=jnp.float32) + b3_ref[...])
    h = lrelu(jnp.dot(h, w4_ref[...],
                      preferred_element_type=jnp.float32) + b4_ref[...])
    h = jnp.dot(h, w5_ref[...],
                preferred_element_type=jnp.float32) + b5_ref[...]
    hm = jnp.mean(h.reshape(64, 64, 32), axis=1)
    logits_ref[...] = jnp.dot(hm, fcw_ref[...],
                              preferred_element_type=jnp.float32) + fcb_ref[...]


def _run_classifier(x, cls):
    x48 = x.reshape(64, 3, 8, 4, 8, 4).transpose(0, 2, 4, 3, 5, 1).reshape(4096, 48)
    w1 = cls['w1'].transpose(2, 3, 1, 0).reshape(48, 128)
    fcw = cls['fc_w'].T
    args = (x48, w1, cls['b1'][None, :], cls['w2'][:, :, 0, 0].T,
            cls['b2'][None, :], cls['w3'][:, :, 0, 0].T, cls['b3'][None, :],
            cls['w4'][:, :, 0, 0].T, cls['b4'][None, :],
            cls['w5'][:, :, 0, 0].T, cls['b5'][None, :], fcw,
            cls['fc_b'][None, :])
    return pl.pallas_call(
        _classifier_body,
        out_shape=jax.ShapeDtypeStruct((64, 3), jnp.float32),
    )(*args)


PB = 4


def _shift_stack(h, k, masks):
    r = k // 2
    pieces = []
    t = 0
    for dy in range(-r, r + 1):
        for dx in range(-r, r + 1):
            off = dy * 32 + dx
            s = jnp.roll(h, -off, axis=1) if off else h
            if dy or dx:
                s = s * masks[t][None, :]
            pieces.append(s)
            t += 1
    return jnp.concatenate(pieces, axis=0)


def _fsrcnn_body(sel_ref, val_ref, x3_ref, m5_ref, m3_ref,
                 wh_ref, bh_ref, ah_ref, ws_ref, bs_ref, as_ref,
                 wm_ref, bm_ref, am_ref, we_ref, be_ref, ae_ref,
                 wd_ref, bd_ref, out_ref):
    i = pl.program_id(0)
    m5 = m5_ref[...]
    m3 = m3_ref[...]
    m3b = m3.astype(jnp.bfloat16)

    def mm(w, im):
        return jnp.dot(w, im, preferred_element_type=jnp.float32)

    def prelu(h, b, a):
        h = h + b
        return jnp.where(h >= 0, h, a * h)

    for j in range(PB):
        p = i * PB + j
        e = sel_ref[p]
        x75 = _shift_stack(x3_ref[j], 5, m5)
        h = prelu(mm(wh_ref[e], x75), bh_ref[e], ah_ref[e])
        h = prelu(mm(ws_ref[e], h), bs_ref[e], as_ref[e])
        for l in range(4):
            im = _shift_stack(h, 3, m3)
            h = prelu(mm(wm_ref[e, l], im), bm_ref[e, l], am_ref[e, l])
        h = prelu(mm(we_ref[e], h), be_ref[e], ae_ref[e])
        im = _shift_stack(h.astype(jnp.bfloat16), 3, m3b)
        y = mm(wd_ref[e], im) + bd_ref[e]
        v = val_ref[p].astype(jnp.float32)
        out_ref[j] = y * v


def _make_masks():
    yy, xx = np.mgrid[0:32, 0:32]
    def mk(k):
        r = k // 2
        ms = []
        for dy in range(-r, r + 1):
            for dx in range(-r, r + 1):
                ok = ((yy + dy >= 0) & (yy + dy < 32) &
                      (xx + dx >= 0) & (xx + dx < 32))
                ms.append(ok.reshape(-1))
        return jnp.asarray(np.stack(ms).astype(np.float32))
    return mk(5), mk(3)


def _expert_offsets(d):
    order = [('w_head', d * 75), ('b_head', d), ('a_head', d),
             ('w_shrink', 12 * d), ('b_shrink', 12), ('a_shrink', 12)]
    for l in range(4):
        order.append(('w_map%d' % l, 1296))
    for l in range(4):
        order.append(('b_map%d' % l, 12))
    for l in range(4):
        order.append(('a_map%d' % l, 12))
    order += [('w_expand', d * 12), ('b_expand', d), ('a_expand', d),
              ('w_deconv', 3 * d * 81), ('b_deconv', 3)]
    offs, t = {}, 0
    for name, sz in order:
        offs[name] = t
        t += sz
    return offs, t


def _pack_indices():
    d_list = (16, 36, 56)
    lmax = max(_expert_offsets(d)[1] for d in d_list)
    ZERO = lmax
    per_type = {
        'Wh': np.zeros((3, D, 75), np.int32),
        'Bh': np.zeros((3, D, 1), np.int32),
        'Ah': np.zeros((3, D, 1), np.int32),
        'Ws': np.zeros((3, S, D), np.int32),
        'Bs': np.zeros((3, S, 1), np.int32),
        'As': np.zeros((3, S, 1), np.int32),
        'Wm': np.zeros((3, 4, S, 9 * S), np.int32),
        'Bm': np.zeros((3, 4, S, 1), np.int32),
        'Am': np.zeros((3, 4, S, 1), np.int32),
        'We': np.zeros((3, D, S), np.int32),
        'Be': np.zeros((3, D, 1), np.int32),
        'Ae': np.zeros((3, D, 1), np.int32),
        'Wd': np.zeros((3, 48, 9 * D), np.int32),
        'Bd': np.zeros((3, 48, 1), np.int32),
    }
    for e, d in enumerate(d_list):
        offs, _ = _expert_offsets(d)
        r56 = np.arange(D)[:, None]
        lane = np.arange(75)[None, :]
        t_l, c_l = lane // 3, lane % 3
        ky, kx = t_l // 5, t_l % 5
        per_type['Wh'][e] = np.where(
            r56 < d, offs['w_head'] + r56 * 75 + c_l * 25 + ky * 5 + kx, ZERO)
        per_type['Bh'][e] = np.where(r56 < d, offs['b_head'] + r56, ZERO)
        per_type['Ah'][e] = np.where(r56 < d, offs['a_head'] + r56, ZERO)
        o12 = np.arange(S)[:, None]
        c56 = np.arange(D)[None, :]
        per_type['Ws'][e] = np.where(
            c56 < d, offs['w_shrink'] + o12 * d + c56, ZERO)
        per_type['Bs'][e] = offs['b_shrink'] + o12
        per_type['As'][e] = offs['a_shrink'] + o12
        lane = np.arange(9 * S)[None, :]
        t_l, c_l = lane // S, lane % S
        for l in range(4):
            per_type['Wm'][e, l] = (offs['w_map%d' % l]
                                    + o12 * 108 + c_l * 9 + t_l)
            per_type['Bm'][e, l] = offs['b_map%d' % l] + o12
            per_type['Am'][e, l] = offs['a_map%d' % l] + o12
        c12 = np.arange(S)[None, :]
        per_type['We'][e] = np.where(
            r56 < d, offs['w_expand'] + r56 * 12 + c12, ZERO)
        per_type['Be'][e] = np.where(r56 < d, offs['b_expand'] + r56, ZERO)
        per_type['Ae'][e] = np.where(r56 < d, offs['a_expand'] + r56, ZERO)
        wd = np.full((48, 9 * D), ZERO, np.int64)
        for ry in range(4):
            for rx in range(4):
                for o in range(3):
                    ch = (ry * 4 + rx) * 3 + o
                    for dy in (-1, 0, 1):
                        for dx in (-1, 0, 1):
                            kyv, kxv = 4 * dy + 6 - ry, 4 * dx + 6 - rx
                            if not (0 <= kyv < 9 and 0 <= kxv < 9):
                                continue
                            t = (dy + 1) * 3 + (dx + 1)
                            c = np.arange(d)
                            wd[ch, t * D + c] = (offs['w_deconv']
                                                 + (o * d + c) * 81
                                                 + kyv * 9 + kxv)
        per_type['Wd'][e] = wd
        per_type['Bd'][e] = (offs['b_deconv']
                             + np.arange(48)[:, None] % 3)
    return per_type, lmax


_PIDX, _LMAX = _pack_indices()


def _flatten_net(p):
    leaves = ([p['w_head'], p['b_head'], p['a_head'],
               p['w_shrink'], p['b_shrink'], p['a_shrink']]
              + list(p['w_map']) + list(p['b_map']) + list(p['a_map'])
              + [p['w_expand'], p['b_expand'], p['a_expand'],
                 p['w_deconv'], p['b_deconv']])
    return jnp.concatenate([l.reshape(-1) for l in leaves])


def _pack_expert_params(nets):
    flats = [_flatten_net(p) for p in nets]
    raw = jnp.stack([jnp.pad(f, (0, _LMAX + 1 - f.shape[0])) for f in flats])
    e_i = np.arange(3)
    def g(name, extra_dims):
        idx = _PIDX[name]
        e = e_i.reshape((3,) + (1,) * extra_dims)
        return raw[e, idx]
    Wh = g('Wh', 2)
    Bh, Ah = g('Bh', 2), g('Ah', 2)
    Ws, Bs, As = g('Ws', 2), g('Bs', 2), g('As', 2)
    Wm, Bm, Am = g('Wm', 3), g('Bm', 3), g('Am', 3)
    We, Be, Ae = g('We', 2), g('Be', 2), g('Ae', 2)
    Wd = g('Wd', 2).astype(jnp.bfloat16)
    Bd = g('Bd', 2)
    return (Wh, Bh, Ah, Ws, Bs, As, Wm, Bm, Am, We, Be, Ae, Wd, Bd)


def _run_fsrcnn(x3, sel, valid, packed, m5, m3):
    (Wh, Bh, Ah, Ws, Bs, As, Wm, Bm, Am, We, Be, Ae, Wd, Bd) = packed

    def full(shape):
        zeros = (0,) * len(shape)
        return pl.BlockSpec(shape, lambda i, sel_ref, val_ref, z=zeros: z)

    grid_spec = pltpu.PrefetchScalarGridSpec(
        num_scalar_prefetch=2,
        grid=(64 // PB,),
        in_specs=[
            pl.BlockSpec((PB, 3, 1024), lambda i, s, v: (i, 0, 0)),
            full(m5.shape), full(m3.shape),
            full(Wh.shape), full(Bh.shape), full(Ah.shape),
            full(Ws.shape), full(Bs.shape), full(As.shape),
            full(Wm.shape), full(Bm.shape), full(Am.shape),
            full(We.shape), full(Be.shape), full(Ae.shape),
            full(Wd.shape), full(Bd.shape),
        ],
        out_specs=pl.BlockSpec((PB, 48, 1024), lambda i, s, v: (i, 0, 0)),
    )
    return pl.pallas_call(
        _fsrcnn_body,
        grid_spec=grid_spec,
        out_shape=jax.ShapeDtypeStruct((64, 48, 1024), jnp.float32),
    )(sel, valid, x3, m5, m3, Wh, Bh, Ah, Ws, Bs, As, Wm, Bm, Am,
      We, Be, Ae, Wd, Bd)


def kernel(x, params):
    logits = _run_classifier(x, params['cls'])

    expert = jnp.argmax(logits, axis=-1).astype(jnp.int32)
    onehot = (expert[:, None] == jnp.arange(3, dtype=jnp.int32)).astype(jnp.int32)
    ranks = jnp.cumsum(onehot, axis=0)
    caps = jnp.asarray(CAPS, jnp.int32)
    myrank = jnp.sum(ranks * onehot, axis=1)
    valid = (myrank <= caps[expert]).astype(jnp.int32)
    counts = jnp.minimum(ranks[-1], caps)

    packed = _pack_expert_params(params['nets'])
    m5, m3 = _make_masks()
    x3 = x.reshape(64, 3, 1024)
    y48 = _run_fsrcnn(x3, expert, valid, packed, m5, m3)

    y = y48.reshape(64, 4, 4, 3, 32, 32)
    y = y.transpose(0, 3, 4, 1, 5, 2).reshape(64, 3, 128, 128)
    return y, counts
